# Initial kernel scaffold; baseline (speedup 1.0000x reference)
import jax
import jax.numpy as jnp
from jax.experimental import pallas as pl


def kernel(x, pos_table):
    raise NotImplementedError("write your pallas kernel here")


# speedup vs baseline: 1.7116x; 1.7116x over previous
import jax
import jax.numpy as jnp
from jax.experimental import pallas as pl

_BT = 128


def _body(x_ref, pos_ref, o_ref):
    o_ref[...] = x_ref[...] + pos_ref[...][None, :, :]


def kernel(x, pos_table):
    b, t, d = x.shape
    bt = _BT if t % _BT == 0 else t
    return pl.pallas_call(
        _body,
        grid=(t // bt,),
        in_specs=[
            pl.BlockSpec((b, bt, d), lambda i: (0, i, 0)),
            pl.BlockSpec((bt, d), lambda i: (i, 0)),
        ],
        out_specs=pl.BlockSpec((b, bt, d), lambda i: (0, i, 0)),
        out_shape=jax.ShapeDtypeStruct((b, t, d), x.dtype),
    )(x, pos_table[:t])


# speedup vs baseline: 1.7142x; 1.0015x over previous
import jax
import jax.numpy as jnp
from jax.experimental import pallas as pl

_BT = 256


def _body(x_ref, pos_ref, o_ref):
    o_ref[...] = x_ref[...] + pos_ref[...][None, :, :]


def kernel(x, pos_table):
    b, t, d = x.shape
    bt = _BT if t % _BT == 0 else t
    return pl.pallas_call(
        _body,
        grid=(t // bt,),
        in_specs=[
            pl.BlockSpec((b, bt, d), lambda i: (0, i, 0)),
            pl.BlockSpec((bt, d), lambda i: (i, 0)),
        ],
        out_specs=pl.BlockSpec((b, bt, d), lambda i: (0, i, 0)),
        out_shape=jax.ShapeDtypeStruct((b, t, d), x.dtype),
    )(x, pos_table[:t])
